# baseline (speedup 1.0000x reference)
import jax
import jax.numpy as jnp
from jax import lax
from jax.experimental import pallas as pl
from jax.experimental.pallas import tpu as pltpu
from jax.experimental.pallas import tpu_sc as plsc

_N = 10000
_E = 320000
_D = 128
_G = 64
_BLK = 128
_NP = 10240
_NB = _NP // _BLK
_NC = 2
_NS = 16
_NW = _NC * _NS
_K = 128
_NCH = _E // _K
_NCHP = 2560
_CPW = _NCHP // _NW
_SINK = _N
_RPT = _NP // _NS
_NBUF = 2
_HCH = _CPW // 2


def _sc_mesh():
  return plsc.VectorSubcoreMesh(core_axis_name="c", subcore_axis_name="s",
                                num_cores=_NC, num_subcores=_NS)


def _make_agg():
  out_type = jax.ShapeDtypeStruct((_NC * _NP, _D), jnp.float32)
  scratch = [
      pltpu.VMEM((_HCH, _K), jnp.int32),
      pltpu.VMEM((_HCH, _K), jnp.int32),
      pltpu.VMEM((_NBUF, _K, _D), jnp.float32),
      pltpu.VMEM_SHARED((_NP, _D), jnp.float32),
  ] + [pltpu.SemaphoreType.DMA] * (2 * _NBUF)

  def body(h_hbm, src_hbm, dst_hbm, zrow_hbm, acc_out, srcv, dstv, msgv,
           accs, *sems):
    gsem = sems[:_NBUF]
    ssem = sems[_NBUF:]
    cid = lax.axis_index("c")
    sid = lax.axis_index("s")
    wid = sid * _NC + cid
    row0 = sid * _RPT

    pltpu.sync_copy(zrow_hbm, accs.at[pl.ds(row0, _RPT)])
    plsc.subcore_barrier()

    def gather(c, b):
      pltpu.async_copy(h_hbm.at[srcv.at[c]], msgv.at[b], gsem[b])

    def gather_wait(c, b):
      pltpu.make_async_copy(h_hbm.at[srcv.at[c]], msgv.at[b], gsem[b]).wait()

    def scatter(c, b):
      pltpu.async_copy(msgv.at[b], accs.at[dstv.at[c]], ssem[b], add=True)

    def scatter_wait(c, b):
      pltpu.make_async_copy(msgv.at[b], accs.at[dstv.at[c]], ssem[b]).wait()

    for half in range(2):
      base = wid * _CPW + half * _HCH
      pltpu.sync_copy(src_hbm.at[pl.ds(base, _HCH)], srcv)
      pltpu.sync_copy(dst_hbm.at[pl.ds(base, _HCH)], dstv)

      for b in range(_NBUF):
        gather(b, b)

      @pl.loop(0, _HCH, step=_NBUF)
      def _(j):
        for b in range(_NBUF):
          gather_wait(j + b, b)
          scatter(j + b, b)
        for b in range(_NBUF):
          scatter_wait(j + b, b)
          nxt = j + b + _NBUF

          @pl.when(nxt < _HCH)
          def _():
            gather(nxt, b)

    plsc.subcore_barrier()
    out_row0 = cid * _NP + row0
    pltpu.sync_copy(accs.at[pl.ds(row0, _RPT)],
                    acc_out.at[pl.ds(out_row0, _RPT)])

  return pl.kernel(body, out_type=out_type, mesh=_sc_mesh(),
                   scratch_types=scratch)


def _make_count():
  out_type = jax.ShapeDtypeStruct((_NC * _NP, _D), jnp.float32)
  scratch = [
      pltpu.VMEM((_CPW, _K), jnp.int32),
      pltpu.VMEM((_K, _D), jnp.float32),
      pltpu.VMEM_SHARED((_NP, _D), jnp.float32),
  ] + [pltpu.SemaphoreType.DMA] * _NBUF

  def body(dst_hbm, ones_hbm, zrow_hbm, cnt_out, dstv, onesv, accs, *ssem):
    cid = lax.axis_index("c")
    sid = lax.axis_index("s")
    wid = sid * _NC + cid
    row0 = sid * _RPT

    pltpu.sync_copy(zrow_hbm, accs.at[pl.ds(row0, _RPT)])
    pltpu.sync_copy(ones_hbm, onesv)
    base = wid * _CPW
    pltpu.sync_copy(dst_hbm.at[pl.ds(base, _CPW)], dstv)
    plsc.subcore_barrier()

    def scatter(c, b):
      pltpu.async_copy(onesv, accs.at[dstv.at[c]], ssem[b], add=True)

    def scatter_wait(c, b):
      pltpu.make_async_copy(onesv, accs.at[dstv.at[c]], ssem[b]).wait()

    @pl.loop(0, _CPW, step=_NBUF)
    def _(j):
      for b in range(_NBUF):
        @pl.when(j > 0)
        def _():
          scatter_wait(j + b - _NBUF, b)
        scatter(j + b, b)

    for b in range(_NBUF):
      scatter_wait(_CPW - _NBUF + b, b)

    plsc.subcore_barrier()
    out_row0 = cid * _NP + row0
    pltpu.sync_copy(accs.at[pl.ds(row0, _RPT)],
                    cnt_out.at[pl.ds(out_row0, _RPT)])

  return pl.kernel(body, out_type=out_type, mesh=_sc_mesh(),
                   scratch_types=scratch)


_AGG_CACHE = {}


def _get_agg(name):
  if name not in _AGG_CACHE:
    _AGG_CACHE[name] = _make_agg() if name == "agg" else _make_count()
  return _AGG_CACHE[name]


def _mm(a, w):
  return lax.dot_general(a, w, (((1,), (1,)), ((), ())),
                         preferred_element_type=jnp.float32,
                         precision=lax.Precision.HIGHEST)


def _lrelu(v):
  return jnp.where(v > 0, v, 0.01 * v)


def _pre_body(x_ref, w_ref, b_ref, o_ref):
  o_ref[...] = _mm(x_ref[...], w_ref[...]) + b_ref[...]


_pre = pl.pallas_call(
    _pre_body,
    grid=(_NB,),
    in_specs=[pl.BlockSpec((_BLK, _D), lambda i: (i, 0)),
              pl.BlockSpec((_D, _D), lambda i: (0, 0)),
              pl.BlockSpec((1, _D), lambda i: (0, 0))],
    out_specs=pl.BlockSpec((_BLK, _D), lambda i: (i, 0)),
    out_shape=jax.ShapeDtypeStruct((_NP, _D), jnp.float32),
)


def _layer_body(a0, a1, c0, c1, h_ref, wl_ref, bl_ref, wr_ref, wh_ref,
                bh_ref, o_ref):
  mean = _combine_mean(a0, a1, c0, c1)
  s = _mm(mean, wl_ref[...]) + bl_ref[...] + _mm(h_ref[...], wr_ref[...])
  s = jnp.maximum(s, 0.0)
  o_ref[...] = _lrelu(_mm(s, wh_ref[...]) + bh_ref[...])


def _make_layer():
  return pl.pallas_call(
      _layer_body,
      grid=(_NB,),
      in_specs=[pl.BlockSpec((_BLK, _D), lambda i: (i, 0)),
                pl.BlockSpec((_BLK, _D), lambda i: (_NB + i, 0)),
                pl.BlockSpec((_BLK, 16), lambda i: (i, 0)),
                pl.BlockSpec((_BLK, 16), lambda i: (_NB + i, 0)),
                pl.BlockSpec((_BLK, _D), lambda i: (i, 0)),
                pl.BlockSpec((_D, _D), lambda i: (0, 0)),
                pl.BlockSpec((1, _D), lambda i: (0, 0)),
                pl.BlockSpec((_D, _D), lambda i: (0, 0)),
                pl.BlockSpec((_D, _D), lambda i: (0, 0)),
                pl.BlockSpec((1, _D), lambda i: (0, 0))],
      out_specs=pl.BlockSpec((_BLK, _D), lambda i: (i, 0)),
      out_shape=jax.ShapeDtypeStruct((_NP, _D), jnp.float32),
  )


_layer1 = _make_layer()
_layer2 = _make_layer()


def _combine_mean(a0, a1, c0, c1):
  cnt = jnp.maximum(c0[...][:, :1] + c1[...][:, :1], 1.0)
  return (a0[...][:, :_D] + a1[...][:, :_D]) / cnt


def _l3_body(a0, a1, c0, c1, h_ref, b_ref, wl_ref, bl_ref, wr_ref, wo_ref,
             bo_ref, ps_ref, gc_ref):
  @pl.when(pl.program_id(0) == 0)
  def _():
    ps_ref[...] = jnp.zeros_like(ps_ref)
    gc_ref[...] = jnp.zeros_like(gc_ref)

  mean = _combine_mean(a0, a1, c0, c1)
  s = _mm(mean, wl_ref[...]) + bl_ref[...] + _mm(h_ref[...], wr_ref[...])
  s = jnp.maximum(s, 0.0)
  t = _lrelu(_mm(s, wo_ref[...]) + bo_ref[...])
  oh = (b_ref[0] == lax.broadcasted_iota(jnp.int32, (_G, 1), 0)
        ).astype(jnp.float32)
  ps_ref[...] += lax.dot_general(oh, t, (((1,), (0,)), ((), ())),
                                 preferred_element_type=jnp.float32,
                                 precision=lax.Precision.HIGHEST)
  gc_ref[...] += jnp.sum(oh, axis=1, keepdims=True)


_layer3_pool = pl.pallas_call(
    _l3_body,
    grid=(_NB,),
    in_specs=[pl.BlockSpec((_BLK, _D), lambda i: (i, 0)),
              pl.BlockSpec((_BLK, _D), lambda i: (_NB + i, 0)),
              pl.BlockSpec((_BLK, 16), lambda i: (i, 0)),
              pl.BlockSpec((_BLK, 16), lambda i: (_NB + i, 0)),
              pl.BlockSpec((_BLK, _D), lambda i: (i, 0)),
              pl.BlockSpec((1, 1, _BLK), lambda i: (i, 0, 0)),
              pl.BlockSpec((_D, _D), lambda i: (0, 0)),
              pl.BlockSpec((1, _D), lambda i: (0, 0)),
              pl.BlockSpec((_D, _D), lambda i: (0, 0)),
              pl.BlockSpec((_D, _D), lambda i: (0, 0)),
              pl.BlockSpec((1, _D), lambda i: (0, 0))],
    out_specs=[pl.BlockSpec((_G, _D), lambda i: (0, 0)),
               pl.BlockSpec((_G, _D), lambda i: (0, 0))],
    out_shape=[jax.ShapeDtypeStruct((_G, _D), jnp.float32),
               jax.ShapeDtypeStruct((_G, _D), jnp.float32)],
)


def _head_body(ps_ref, gc_ref, woh_ref, boh_ref, gam_ref, bet_ref, rm_ref,
               rv_ref, wh1_ref, bh1_ref, o_ref):
  pooled = ps_ref[...] / jnp.maximum(gc_ref[...][:, :1], 1.0)
  hh = _mm(pooled, woh_ref[...]) + boh_ref[...]
  hh = (hh - rm_ref[...]) / jnp.sqrt(rv_ref[...] + 1e-5) * gam_ref[...] \
      + bet_ref[...]
  hh = _lrelu(hh)
  hw = jnp.sum(hh * wh1_ref[...], axis=1, keepdims=True)
  o_ref[...] = jnp.maximum(hw + bh1_ref[0, 0], 0.0)


_head = pl.pallas_call(
    _head_body,
    out_shape=jax.ShapeDtypeStruct((_G, 1), jnp.float32),
)


def kernel(x, edge_index, batch, W_pre, b_pre, Wl1, bl1, Wr1, Wl2, bl2, Wr2,
           Wl3, bl3, Wr3, W_hh1, b_hh1, W_hh2, b_hh2, W_oo, b_oo,
           W_oh, b_oh, gamma_h, beta_h, rm_h, rv_h, W_h1, b_h1):
  npad = (_NCHP - _NCH) * _K
  src = jnp.pad(edge_index[0], (0, npad)).reshape(_NCHP, _K)
  dst = jnp.pad(edge_index[1], (0, npad),
                constant_values=_SINK).reshape(_NCHP, _K)
  xp = jnp.pad(x, ((0, _NP - _N), (0, 0)))
  batch_p = jnp.pad(batch, (0, _NP - _N),
                    constant_values=_G).reshape(_NB, 1, _BLK)
  zrow = jnp.zeros((_RPT, _D), jnp.float32)
  ones = jnp.ones((_K, _D), jnp.float32)

  r1 = lambda v: v.reshape(1, -1)

  h0 = _pre(xp, W_pre, r1(b_pre))
  cnt = _get_agg("count")(dst, ones, zrow)[:, :16]
  acc1 = _get_agg("agg")(h0, src, dst, zrow)
  h1 = _layer1(acc1, acc1, cnt, cnt, h0, Wl1, r1(bl1), Wr1, W_hh1, r1(b_hh1))
  acc2 = _get_agg("agg")(h1, src, dst, zrow)
  h2 = _layer2(acc2, acc2, cnt, cnt, h1, Wl2, r1(bl2), Wr2, W_hh2, r1(b_hh2))
  acc3 = _get_agg("agg")(h2, src, dst, zrow)
  psum, gcnt = _layer3_pool(acc3, acc3, cnt, cnt, h2, batch_p,
                            Wl3, r1(bl3), Wr3, W_oo, r1(b_oo))
  return _head(psum, gcnt, W_oh, r1(b_oh), r1(gamma_h), r1(beta_h),
               r1(rm_h), r1(rv_h), W_h1, b_h1.reshape(1, 1))


# speedup vs baseline: 2.1951x; 2.1951x over previous
import jax
import jax.numpy as jnp
from jax import lax
from jax.experimental import pallas as pl
from jax.experimental.pallas import tpu as pltpu
from jax.experimental.pallas import tpu_sc as plsc

_N = 10000
_E = 320000
_D = 128
_G = 64
_BLK = 128
_NP = 10240
_NB = _NP // _BLK
_NC = 2
_NS = 16
_NW = _NC * _NS
_K = 128
_NCH = _E // _K
_NCHP = 2560
_CPW = _NCHP // _NW
_SINK = _N
_RPT = _NP // _NS
_NBUF = 2
_HCH = _CPW // 2


def _sc_mesh():
  return plsc.VectorSubcoreMesh(core_axis_name="c", subcore_axis_name="s",
                                num_cores=_NC, num_subcores=_NS)


def _make_agg():
  out_type = jax.ShapeDtypeStruct((_NC * _NP, _D), jnp.float32)
  scratch = [
      pltpu.VMEM((_HCH, _K), jnp.int32),
      pltpu.VMEM((_HCH, _K), jnp.int32),
      pltpu.VMEM((_NBUF, _K, _D), jnp.float32),
      pltpu.VMEM_SHARED((_NP, _D), jnp.float32),
  ] + [pltpu.SemaphoreType.DMA] * (2 * _NBUF)

  def body(h_hbm, src_hbm, dst_hbm, zrow_hbm, acc_out, srcv, dstv, msgv,
           accs, *sems):
    gsem = sems[:_NBUF]
    ssem = sems[_NBUF:]
    cid = lax.axis_index("c")
    sid = lax.axis_index("s")
    wid = sid * _NC + cid
    row0 = sid * _RPT

    pltpu.sync_copy(zrow_hbm, accs.at[pl.ds(row0, _RPT)])
    plsc.subcore_barrier()

    def gather(c, b):
      pltpu.async_copy(h_hbm.at[srcv.at[c]], msgv.at[b], gsem[b])

    def gather_wait(c, b):
      pltpu.make_async_copy(h_hbm.at[srcv.at[c]], msgv.at[b], gsem[b]).wait()

    def scatter(c, b):
      pltpu.async_copy(msgv.at[b], accs.at[dstv.at[c]], ssem[b], add=True)

    def scatter_wait(c, b):
      pltpu.make_async_copy(msgv.at[b], accs.at[dstv.at[c]], ssem[b]).wait()

    for half in range(2):
      base = wid * _CPW + half * _HCH
      pltpu.sync_copy(src_hbm.at[pl.ds(base, _HCH)], srcv)
      pltpu.sync_copy(dst_hbm.at[pl.ds(base, _HCH)], dstv)

      for b in range(_NBUF):
        gather(b, b)

      @pl.loop(0, _HCH, step=_NBUF)
      def _(j):
        for b in range(_NBUF):
          gather_wait(j + b, b)
          scatter(j + b, b)
        for b in range(_NBUF):
          scatter_wait(j + b, b)
          nxt = j + b + _NBUF

          @pl.when(nxt < _HCH)
          def _():
            gather(nxt, b)

    plsc.subcore_barrier()
    out_row0 = cid * _NP + row0
    pltpu.sync_copy(accs.at[pl.ds(row0, _RPT)],
                    acc_out.at[pl.ds(out_row0, _RPT)])

  return pl.kernel(body, out_type=out_type, mesh=_sc_mesh(),
                   scratch_types=scratch)


def _make_count():
  out_type = jax.ShapeDtypeStruct((_NC * _NP, _D), jnp.float32)
  scratch = [
      pltpu.VMEM((_CPW, _K), jnp.int32),
      pltpu.VMEM((_K, _D), jnp.float32),
      pltpu.VMEM_SHARED((_NP, _D), jnp.float32),
  ] + [pltpu.SemaphoreType.DMA] * _NBUF

  def body(dst_hbm, ones_hbm, zrow_hbm, cnt_out, dstv, onesv, accs, *ssem):
    cid = lax.axis_index("c")
    sid = lax.axis_index("s")
    wid = sid * _NC + cid
    row0 = sid * _RPT

    pltpu.sync_copy(zrow_hbm, accs.at[pl.ds(row0, _RPT)])
    pltpu.sync_copy(ones_hbm, onesv)
    base = wid * _CPW
    pltpu.sync_copy(dst_hbm.at[pl.ds(base, _CPW)], dstv)
    plsc.subcore_barrier()

    def scatter(c, b):
      pltpu.async_copy(onesv, accs.at[dstv.at[c]], ssem[b], add=True)

    def scatter_wait(c, b):
      pltpu.make_async_copy(onesv, accs.at[dstv.at[c]], ssem[b]).wait()

    @pl.loop(0, _CPW, step=_NBUF)
    def _(j):
      for b in range(_NBUF):
        @pl.when(j > 0)
        def _():
          scatter_wait(j + b - _NBUF, b)
        scatter(j + b, b)

    for b in range(_NBUF):
      scatter_wait(_CPW - _NBUF + b, b)

    plsc.subcore_barrier()
    out_row0 = cid * _NP + row0
    pltpu.sync_copy(accs.at[pl.ds(row0, _RPT)],
                    cnt_out.at[pl.ds(out_row0, _RPT)])

  return pl.kernel(body, out_type=out_type, mesh=_sc_mesh(),
                   scratch_types=scratch)


_AGG_CACHE = {}


def _get_agg(name):
  if name not in _AGG_CACHE:
    _AGG_CACHE[name] = _make_agg() if name == "agg" else _make_count()
  return _AGG_CACHE[name]


def _mm(a, w):
  return lax.dot_general(a, w, (((1,), (1,)), ((), ())),
                         preferred_element_type=jnp.float32,
                         precision=lax.Precision.HIGHEST)


def _lrelu(v):
  return jnp.where(v > 0, v, 0.01 * v)


def _pre_body(x_ref, w_ref, b_ref, o_ref):
  o_ref[...] = _mm(x_ref[...], w_ref[...]) + b_ref[...]


_pre = pl.pallas_call(
    _pre_body,
    grid=(_NB,),
    in_specs=[pl.BlockSpec((_BLK, _D), lambda i: (i, 0)),
              pl.BlockSpec((_D, _D), lambda i: (0, 0)),
              pl.BlockSpec((1, _D), lambda i: (0, 0))],
    out_specs=pl.BlockSpec((_BLK, _D), lambda i: (i, 0)),
    out_shape=jax.ShapeDtypeStruct((_NP, _D), jnp.float32),
)


def _layer_body(a0, a1, c0, c1, h_ref, wl_ref, bl_ref, wr_ref, wh_ref,
                bh_ref, o_ref):
  mean = _combine_mean(a0, a1, c0, c1)
  s = _mm(mean, wl_ref[...]) + bl_ref[...] + _mm(h_ref[...], wr_ref[...])
  s = jnp.maximum(s, 0.0)
  o_ref[...] = _lrelu(_mm(s, wh_ref[...]) + bh_ref[...])


def _make_layer():
  return pl.pallas_call(
      _layer_body,
      grid=(_NB,),
      in_specs=[pl.BlockSpec((_BLK, _D), lambda i: (i, 0)),
                pl.BlockSpec((_BLK, _D), lambda i: (_NB + i, 0)),
                pl.BlockSpec((_BLK, 16), lambda i: (i, 0)),
                pl.BlockSpec((_BLK, 16), lambda i: (_NB + i, 0)),
                pl.BlockSpec((_BLK, _D), lambda i: (i, 0)),
                pl.BlockSpec((_D, _D), lambda i: (0, 0)),
                pl.BlockSpec((1, _D), lambda i: (0, 0)),
                pl.BlockSpec((_D, _D), lambda i: (0, 0)),
                pl.BlockSpec((_D, _D), lambda i: (0, 0)),
                pl.BlockSpec((1, _D), lambda i: (0, 0))],
      out_specs=pl.BlockSpec((_BLK, _D), lambda i: (i, 0)),
      out_shape=jax.ShapeDtypeStruct((_NP, _D), jnp.float32),
  )


_layer1 = _make_layer()
_layer2 = _make_layer()


def _combine_mean(a0, a1, c0, c1):
  cnt = jnp.maximum(c0[...][:, :1] + c1[...][:, :1], 1.0)
  return (a0[...][:, :_D] + a1[...][:, :_D]) / cnt


def _l3_body(a0, a1, c0, c1, h_ref, b_ref, wl_ref, bl_ref, wr_ref, wo_ref,
             bo_ref, ps_ref, gc_ref):
  @pl.when(pl.program_id(0) == 0)
  def _():
    ps_ref[...] = jnp.zeros_like(ps_ref)
    gc_ref[...] = jnp.zeros_like(gc_ref)

  mean = _combine_mean(a0, a1, c0, c1)
  s = _mm(mean, wl_ref[...]) + bl_ref[...] + _mm(h_ref[...], wr_ref[...])
  s = jnp.maximum(s, 0.0)
  t = _lrelu(_mm(s, wo_ref[...]) + bo_ref[...])
  oh = (b_ref[0] == lax.broadcasted_iota(jnp.int32, (_G, 1), 0)
        ).astype(jnp.float32)
  ps_ref[...] += lax.dot_general(oh, t, (((1,), (0,)), ((), ())),
                                 preferred_element_type=jnp.float32,
                                 precision=lax.Precision.HIGHEST)
  gc_ref[...] += jnp.sum(oh, axis=1, keepdims=True)


_layer3_pool = pl.pallas_call(
    _l3_body,
    grid=(_NB,),
    in_specs=[pl.BlockSpec((_BLK, _D), lambda i: (i, 0)),
              pl.BlockSpec((_BLK, _D), lambda i: (_NB + i, 0)),
              pl.BlockSpec((_BLK, 16), lambda i: (i, 0)),
              pl.BlockSpec((_BLK, 16), lambda i: (_NB + i, 0)),
              pl.BlockSpec((_BLK, _D), lambda i: (i, 0)),
              pl.BlockSpec((1, 1, _BLK), lambda i: (i, 0, 0)),
              pl.BlockSpec((_D, _D), lambda i: (0, 0)),
              pl.BlockSpec((1, _D), lambda i: (0, 0)),
              pl.BlockSpec((_D, _D), lambda i: (0, 0)),
              pl.BlockSpec((_D, _D), lambda i: (0, 0)),
              pl.BlockSpec((1, _D), lambda i: (0, 0))],
    out_specs=[pl.BlockSpec((_G, _D), lambda i: (0, 0)),
               pl.BlockSpec((_G, _D), lambda i: (0, 0))],
    out_shape=[jax.ShapeDtypeStruct((_G, _D), jnp.float32),
               jax.ShapeDtypeStruct((_G, _D), jnp.float32)],
)


def _head_body(ps_ref, gc_ref, woh_ref, boh_ref, gam_ref, bet_ref, rm_ref,
               rv_ref, wh1_ref, bh1_ref, o_ref):
  pooled = ps_ref[...] / jnp.maximum(gc_ref[...][:, :1], 1.0)
  hh = _mm(pooled, woh_ref[...]) + boh_ref[...]
  hh = (hh - rm_ref[...]) / jnp.sqrt(rv_ref[...] + 1e-5) * gam_ref[...] \
      + bet_ref[...]
  hh = _lrelu(hh)
  hw = jnp.sum(hh * wh1_ref[...], axis=1, keepdims=True)
  o_ref[...] = jnp.maximum(hw + bh1_ref[0, 0], 0.0)


_head = pl.pallas_call(
    _head_body,
    out_shape=jax.ShapeDtypeStruct((_G, 1), jnp.float32),
)


def kernel(x, edge_index, batch, W_pre, b_pre, Wl1, bl1, Wr1, Wl2, bl2, Wr2,
           Wl3, bl3, Wr3, W_hh1, b_hh1, W_hh2, b_hh2, W_oo, b_oo,
           W_oh, b_oh, gamma_h, beta_h, rm_h, rv_h, W_h1, b_h1):
  npad = (_NCHP - _NCH) * _K
  sink = _SINK + (jnp.arange(npad, dtype=jnp.int32) % (_NP - _N))
  src = jnp.concatenate([edge_index[0], sink]).reshape(_NCHP, _K)
  dst = jnp.concatenate([edge_index[1], sink]).reshape(_NCHP, _K)
  xp = jnp.pad(x, ((0, _NP - _N), (0, 0)))
  batch_p = jnp.pad(batch, (0, _NP - _N),
                    constant_values=_G).reshape(_NB, 1, _BLK)
  zrow = jnp.zeros((_RPT, _D), jnp.float32)
  ones = jnp.ones((_K, _D), jnp.float32)

  r1 = lambda v: v.reshape(1, -1)

  h0 = _pre(xp, W_pre, r1(b_pre))
  cnt = _get_agg("count")(dst, ones, zrow)[:, :16]
  acc1 = _get_agg("agg")(h0, src, dst, zrow)
  h1 = _layer1(acc1, acc1, cnt, cnt, h0, Wl1, r1(bl1), Wr1, W_hh1, r1(b_hh1))
  acc2 = _get_agg("agg")(h1, src, dst, zrow)
  h2 = _layer2(acc2, acc2, cnt, cnt, h1, Wl2, r1(bl2), Wr2, W_hh2, r1(b_hh2))
  acc3 = _get_agg("agg")(h2, src, dst, zrow)
  psum, gcnt = _layer3_pool(acc3, acc3, cnt, cnt, h2, batch_p,
                            Wl3, r1(bl3), Wr3, W_oo, r1(b_oo))
  return _head(psum, gcnt, W_oh, r1(b_oh), r1(gamma_h), r1(beta_h),
               r1(rm_h), r1(rv_h), W_h1, b_h1.reshape(1, 1))


# speedup vs baseline: 3.0768x; 1.4017x over previous
import jax
import jax.numpy as jnp
from jax import lax
from jax.experimental import pallas as pl
from jax.experimental.pallas import tpu as pltpu
from jax.experimental.pallas import tpu_sc as plsc

_N = 10000
_E = 320000
_D = 128
_G = 64
_BLK = 128
_NP = 10240
_NB = _NP // _BLK
_NC = 2
_NS = 16
_NW = _NC * _NS
_K = 128
_NCH = _E // _K
_NCHP = 2560
_CPW = _NCHP // _NW
_SINK = _N
_RPT = _NP // _NS
_NBUF = 2
_HCH = _CPW // 2


def _sc_mesh():
  return plsc.VectorSubcoreMesh(core_axis_name="c", subcore_axis_name="s",
                                num_cores=_NC, num_subcores=_NS)


def _make_agg():
  out_type = jax.ShapeDtypeStruct((_NC * _NP, _D), jnp.float32)
  scratch = [
      pltpu.VMEM((_HCH, _K), jnp.int32),
      pltpu.VMEM((_HCH, _K), jnp.int32),
      pltpu.VMEM((_NBUF, _K, _D), jnp.float32),
      pltpu.VMEM_SHARED((_NP, _D), jnp.float32),
  ] + [pltpu.SemaphoreType.DMA] * (2 * _NBUF)

  def body(h_hbm, src_hbm, dst_hbm, zrow_hbm, acc_out, srcv, dstv, msgv,
           accs, *sems):
    gsem = sems[:_NBUF]
    ssem = sems[_NBUF:]
    cid = lax.axis_index("c")
    sid = lax.axis_index("s")
    wid = sid * _NC + cid
    row0 = sid * _RPT

    pltpu.sync_copy(zrow_hbm, accs.at[pl.ds(row0, _RPT)])
    plsc.subcore_barrier()

    def gather(c, b):
      pltpu.async_copy(h_hbm.at[srcv.at[c]], msgv.at[b], gsem[b])

    def gather_wait(c, b):
      pltpu.make_async_copy(h_hbm.at[srcv.at[c]], msgv.at[b], gsem[b]).wait()

    def scatter(c, b):
      pltpu.async_copy(msgv.at[b], accs.at[dstv.at[c]], ssem[b], add=True)

    def scatter_wait(c, b):
      pltpu.make_async_copy(msgv.at[b], accs.at[dstv.at[c]], ssem[b]).wait()

    for half in range(2):
      base = wid * _CPW + half * _HCH
      pltpu.sync_copy(src_hbm.at[pl.ds(base, _HCH)], srcv)
      pltpu.sync_copy(dst_hbm.at[pl.ds(base, _HCH)], dstv)

      for b in range(_NBUF):
        gather(b, b)

      @pl.loop(0, _HCH, step=_NBUF)
      def _(j):
        for b in range(_NBUF):
          gather_wait(j + b, b)
          scatter(j + b, b)
        for b in range(_NBUF):
          scatter_wait(j + b, b)
          nxt = j + b + _NBUF

          @pl.when(nxt < _HCH)
          def _():
            gather(nxt, b)

    plsc.subcore_barrier()
    out_row0 = cid * _NP + row0
    pltpu.sync_copy(accs.at[pl.ds(row0, _RPT)],
                    acc_out.at[pl.ds(out_row0, _RPT)])

  return pl.kernel(body, out_type=out_type, mesh=_sc_mesh(),
                   scratch_types=scratch)


def _make_count():
  out_type = jax.ShapeDtypeStruct((_NC * _NP, _D), jnp.float32)
  scratch = [
      pltpu.VMEM((_CPW, _K), jnp.int32),
      pltpu.VMEM((_K, _D), jnp.float32),
      pltpu.VMEM_SHARED((_NP, _D), jnp.float32),
  ] + [pltpu.SemaphoreType.DMA] * _NBUF

  def body(dst_hbm, ones_hbm, zrow_hbm, cnt_out, dstv, onesv, accs, *ssem):
    cid = lax.axis_index("c")
    sid = lax.axis_index("s")
    wid = sid * _NC + cid
    row0 = sid * _RPT

    pltpu.sync_copy(zrow_hbm, accs.at[pl.ds(row0, _RPT)])
    pltpu.sync_copy(ones_hbm, onesv)
    base = wid * _CPW
    pltpu.sync_copy(dst_hbm.at[pl.ds(base, _CPW)], dstv)
    plsc.subcore_barrier()

    def scatter(c, b):
      pltpu.async_copy(onesv, accs.at[dstv.at[c]], ssem[b], add=True)

    def scatter_wait(c, b):
      pltpu.make_async_copy(onesv, accs.at[dstv.at[c]], ssem[b]).wait()

    @pl.loop(0, _CPW, step=_NBUF)
    def _(j):
      for b in range(_NBUF):
        @pl.when(j > 0)
        def _():
          scatter_wait(j + b - _NBUF, b)
        scatter(j + b, b)

    for b in range(_NBUF):
      scatter_wait(_CPW - _NBUF + b, b)

    plsc.subcore_barrier()
    out_row0 = cid * _NP + row0
    pltpu.sync_copy(accs.at[pl.ds(row0, _RPT)],
                    cnt_out.at[pl.ds(out_row0, _RPT)])

  return pl.kernel(body, out_type=out_type, mesh=_sc_mesh(),
                   scratch_types=scratch)


_AGG_CACHE = {}


def _get_agg(name):
  if name not in _AGG_CACHE:
    _AGG_CACHE[name] = _make_agg() if name == "agg" else _make_count()
  return _AGG_CACHE[name]


def _mm(a, w):
  return lax.dot_general(a, w, (((1,), (1,)), ((), ())),
                         preferred_element_type=jnp.float32,
                         precision=lax.Precision.DEFAULT)


def _lrelu(v):
  return jnp.where(v > 0, v, 0.01 * v)


_BLKC = 2048
_NBC = _NP // _BLKC


def _wspec(shape=(_D, _D)):
  return pl.BlockSpec(shape, lambda i: tuple(0 for _ in shape))


def _combine_mean(a0, a1, c0, c1):
  cnt = jnp.maximum(c0[...][:, :1] + c1[...][:, :1], 1.0)
  return (a0[...] + a1[...]) / cnt


def _pre_body(x_ref, w_ref, b_ref, wrn_ref, oh_ref, orn_ref):
  h = _mm(x_ref[...], w_ref[...]) + b_ref[...]
  oh_ref[...] = h
  orn_ref[...] = _mm(h, wrn_ref[...])


_pre = pl.pallas_call(
    _pre_body,
    grid=(_NBC,),
    in_specs=[pl.BlockSpec((_BLKC, _D), lambda i: (i, 0)),
              _wspec(), _wspec((1, _D)), _wspec()],
    out_specs=[pl.BlockSpec((_BLKC, _D), lambda i: (i, 0)),
               pl.BlockSpec((_BLKC, _D), lambda i: (i, 0))],
    out_shape=[jax.ShapeDtypeStruct((_NP, _D), jnp.float32),
               jax.ShapeDtypeStruct((_NP, _D), jnp.float32)],
)


def _comb_body(a0, a1, c0, c1, r_ref, wl_ref, bl_ref, wh_ref, bh_ref,
               wrn_ref, oh_ref, orn_ref):
  mean = _combine_mean(a0, a1, c0, c1)
  s = jnp.maximum(_mm(mean, wl_ref[...]) + bl_ref[...] + r_ref[...], 0.0)
  h = _lrelu(_mm(s, wh_ref[...]) + bh_ref[...])
  oh_ref[...] = h
  orn_ref[...] = _mm(h, wrn_ref[...])


def _make_comb():
  return pl.pallas_call(
      _comb_body,
      grid=(_NBC,),
      in_specs=[pl.BlockSpec((_BLKC, _D), lambda i: (i, 0)),
                pl.BlockSpec((_BLKC, _D), lambda i: (_NBC + i, 0)),
                pl.BlockSpec((_BLKC, 16), lambda i: (i, 0)),
                pl.BlockSpec((_BLKC, 16), lambda i: (_NBC + i, 0)),
                pl.BlockSpec((_BLKC, _D), lambda i: (i, 0)),
                _wspec(), _wspec((1, _D)), _wspec(), _wspec((1, _D)),
                _wspec()],
      out_specs=[pl.BlockSpec((_BLKC, _D), lambda i: (i, 0)),
                 pl.BlockSpec((_BLKC, _D), lambda i: (i, 0))],
      out_shape=[jax.ShapeDtypeStruct((_NP, _D), jnp.float32),
                 jax.ShapeDtypeStruct((_NP, _D), jnp.float32)],
  )


_comb1 = _make_comb()
_comb2 = _make_comb()


def _l3_body(a0, a1, c0, c1, r_ref, b_ref, wl_ref, bl_ref, wo_ref, bo_ref,
             woh_ref, boh_ref, gam_ref, bet_ref, rm_ref, rv_ref, wh1_ref,
             bh1_ref, o_ref, ps_ref, gc_ref):
  @pl.when(pl.program_id(0) == 0)
  def _():
    ps_ref[...] = jnp.zeros_like(ps_ref)
    gc_ref[...] = jnp.zeros_like(gc_ref)

  mean = _combine_mean(a0, a1, c0, c1)
  s = jnp.maximum(_mm(mean, wl_ref[...]) + bl_ref[...] + r_ref[...], 0.0)
  t = _lrelu(_mm(s, wo_ref[...]) + bo_ref[...])
  oh = (b_ref[0] == lax.broadcasted_iota(jnp.int32, (_G, 1), 0)
        ).astype(jnp.float32)
  ps_ref[...] += lax.dot_general(oh, t, (((1,), (0,)), ((), ())),
                                 preferred_element_type=jnp.float32,
                                 precision=lax.Precision.HIGHEST)
  gc_ref[...] += jnp.sum(oh, axis=1, keepdims=True)

  @pl.when(pl.program_id(0) == _NBC - 1)
  def _():
    pooled = ps_ref[...] / jnp.maximum(gc_ref[...][:, :1], 1.0)
    hh = _mm(pooled, woh_ref[...]) + boh_ref[...]
    hh = (hh - rm_ref[...]) / jnp.sqrt(rv_ref[...] + 1e-5) * gam_ref[...] \
        + bet_ref[...]
    hh = _lrelu(hh)
    hw = jnp.sum(hh * wh1_ref[...], axis=1, keepdims=True)
    o_ref[...] = jnp.maximum(hw + bh1_ref[0, 0], 0.0)


_layer3_pool_head = pl.pallas_call(
    _l3_body,
    grid=(_NBC,),
    in_specs=[pl.BlockSpec((_BLKC, _D), lambda i: (i, 0)),
              pl.BlockSpec((_BLKC, _D), lambda i: (_NBC + i, 0)),
              pl.BlockSpec((_BLKC, 16), lambda i: (i, 0)),
              pl.BlockSpec((_BLKC, 16), lambda i: (_NBC + i, 0)),
              pl.BlockSpec((_BLKC, _D), lambda i: (i, 0)),
              pl.BlockSpec((1, 1, _BLKC), lambda i: (i, 0, 0)),
              _wspec(), _wspec((1, _D)), _wspec(), _wspec((1, _D)),
              _wspec(), _wspec((1, _D)), _wspec((1, _D)), _wspec((1, _D)),
              _wspec((1, _D)), _wspec((1, _D)), _wspec((1, _D)),
              _wspec((1, 1))],
    out_specs=pl.BlockSpec((_G, 1), lambda i: (0, 0)),
    out_shape=jax.ShapeDtypeStruct((_G, 1), jnp.float32),
    scratch_shapes=[pltpu.VMEM((_G, _D), jnp.float32),
                    pltpu.VMEM((_G, 1), jnp.float32)],
)


def kernel(x, edge_index, batch, W_pre, b_pre, Wl1, bl1, Wr1, Wl2, bl2, Wr2,
           Wl3, bl3, Wr3, W_hh1, b_hh1, W_hh2, b_hh2, W_oo, b_oo,
           W_oh, b_oh, gamma_h, beta_h, rm_h, rv_h, W_h1, b_h1):
  npad = (_NCHP - _NCH) * _K
  sink = _SINK + (jnp.arange(npad, dtype=jnp.int32) % (_NP - _N))
  src = jnp.concatenate([edge_index[0], sink]).reshape(_NCHP, _K)
  dst = jnp.concatenate([edge_index[1], sink]).reshape(_NCHP, _K)
  xp = jnp.pad(x, ((0, _NP - _N), (0, 0)))
  batch_p = jnp.pad(batch, (0, _NP - _N),
                    constant_values=_G).reshape(_NBC, 1, _BLKC)
  zrow = jnp.zeros((_RPT, _D), jnp.float32)
  ones = jnp.ones((_K, _D), jnp.float32)

  r1 = lambda v: v.reshape(1, -1)

  cnt = _get_agg("count")(dst, ones, zrow)[:, :16]
  h0, rn1 = _pre(xp, W_pre, r1(b_pre), Wr1)
  acc1 = _get_agg("agg")(h0, src, dst, zrow)
  h1, rn2 = _comb1(acc1, acc1, cnt, cnt, rn1, Wl1, r1(bl1),
                   W_hh1, r1(b_hh1), Wr2)
  acc2 = _get_agg("agg")(h1, src, dst, zrow)
  h2, rn3 = _comb2(acc2, acc2, cnt, cnt, rn2, Wl2, r1(bl2),
                   W_hh2, r1(b_hh2), Wr3)
  acc3 = _get_agg("agg")(h2, src, dst, zrow)
  return _layer3_pool_head(acc3, acc3, cnt, cnt, rn3, batch_p,
                           Wl3, r1(bl3), W_oo, r1(b_oo),
                           W_oh, r1(b_oh), r1(gamma_h), r1(beta_h),
                           r1(rm_h), r1(rv_h), W_h1, b_h1.reshape(1, 1))


# speedup vs baseline: 3.0969x; 1.0066x over previous
import jax
import jax.numpy as jnp
from jax import lax
from jax.experimental import pallas as pl
from jax.experimental.pallas import tpu as pltpu
from jax.experimental.pallas import tpu_sc as plsc

_N = 10000
_E = 320000
_D = 128
_G = 64
_BLK = 128
_NP = 10240
_NB = _NP // _BLK
_NC = 2
_NS = 16
_NW = _NC * _NS
_K = 128
_NCH = _E // _K
_NCHP = 2560
_CPW = _NCHP // _NW
_SINK = _N
_RPT = _NP // _NS
_NBUF = 2
_HCH = _CPW // 2


def _sc_mesh():
  return plsc.VectorSubcoreMesh(core_axis_name="c", subcore_axis_name="s",
                                num_cores=_NC, num_subcores=_NS)


def _make_agg():
  out_type = jax.ShapeDtypeStruct((_NC * _NP, _D), jnp.float32)
  scratch = [
      pltpu.VMEM((_HCH, _K), jnp.int32),
      pltpu.VMEM((_HCH, _K), jnp.int32),
      pltpu.VMEM((_NBUF, _K, _D), jnp.float32),
      pltpu.VMEM_SHARED((_NP, _D), jnp.float32),
  ] + [pltpu.SemaphoreType.DMA] * (2 * _NBUF)

  def body(h_hbm, src_hbm, dst_hbm, zrow_hbm, cnt_hbm, acc_out, srcv, dstv,
           msgv, accs, *sems):
    del cnt_hbm
    gsem = sems[:_NBUF]
    ssem = sems[_NBUF:]
    cid = lax.axis_index("c")
    sid = lax.axis_index("s")
    wid = sid * _NC + cid
    row0 = sid * _RPT

    pltpu.sync_copy(zrow_hbm, accs.at[pl.ds(row0, _RPT)])
    plsc.subcore_barrier()

    def gather(c, b):
      pltpu.async_copy(h_hbm.at[srcv.at[c]], msgv.at[b], gsem[b])

    def gather_wait(c, b):
      pltpu.make_async_copy(h_hbm.at[srcv.at[c]], msgv.at[b], gsem[b]).wait()

    def scatter(c, b):
      pltpu.async_copy(msgv.at[b], accs.at[dstv.at[c]], ssem[b], add=True)

    def scatter_wait(c, b):
      pltpu.make_async_copy(msgv.at[b], accs.at[dstv.at[c]], ssem[b]).wait()

    for half in range(2):
      base = wid * _CPW + half * _HCH
      pltpu.sync_copy(src_hbm.at[pl.ds(base, _HCH)], srcv)
      pltpu.sync_copy(dst_hbm.at[pl.ds(base, _HCH)], dstv)

      for b in range(_NBUF):
        gather(b, b)

      @pl.loop(0, _HCH, step=_NBUF)
      def _(j):
        for b in range(_NBUF):
          gather_wait(j + b, b)
          scatter(j + b, b)
        for b in range(_NBUF):
          scatter_wait(j + b, b)
          nxt = j + b + _NBUF

          @pl.when(nxt < _HCH)
          def _():
            gather(nxt, b)

    plsc.subcore_barrier()
    out_row0 = cid * _NP + row0
    pltpu.sync_copy(accs.at[pl.ds(row0, _RPT)],
                    acc_out.at[pl.ds(out_row0, _RPT)])

  return pl.kernel(body, out_type=out_type, mesh=_sc_mesh(),
                   scratch_types=scratch)


def _make_count():
  out_type = jax.ShapeDtypeStruct((_NC * _NP, _D), jnp.float32)
  scratch = [
      pltpu.VMEM((_CPW, _K), jnp.int32),
      pltpu.VMEM((_K, _D), jnp.float32),
      pltpu.VMEM_SHARED((_NP, _D), jnp.float32),
  ] + [pltpu.SemaphoreType.DMA] * _NBUF

  def body(dst_hbm, ones_hbm, zrow_hbm, cnt_out, dstv, onesv, accs, *ssem):
    cid = lax.axis_index("c")
    sid = lax.axis_index("s")
    wid = sid * _NC + cid
    row0 = sid * _RPT

    pltpu.sync_copy(zrow_hbm, accs.at[pl.ds(row0, _RPT)])
    pltpu.sync_copy(ones_hbm, onesv)
    base = wid * _CPW
    pltpu.sync_copy(dst_hbm.at[pl.ds(base, _CPW)], dstv)
    plsc.subcore_barrier()

    def scatter(c, b):
      pltpu.async_copy(onesv, accs.at[dstv.at[c]], ssem[b], add=True)

    def scatter_wait(c, b):
      pltpu.make_async_copy(onesv, accs.at[dstv.at[c]], ssem[b]).wait()

    @pl.loop(0, _CPW, step=_NBUF)
    def _(j):
      for b in range(_NBUF):
        @pl.when(j > 0)
        def _():
          scatter_wait(j + b - _NBUF, b)
        scatter(j + b, b)

    for b in range(_NBUF):
      scatter_wait(_CPW - _NBUF + b, b)

    plsc.subcore_barrier()
    out_row0 = cid * _NP + row0
    pltpu.sync_copy(accs.at[pl.ds(row0, _RPT)],
                    cnt_out.at[pl.ds(out_row0, _RPT)])

  return pl.kernel(body, out_type=out_type, mesh=_sc_mesh(),
                   scratch_types=scratch)


_AGG_CACHE = {}


def _get_agg(name):
  if name not in _AGG_CACHE:
    _AGG_CACHE[name] = _make_agg() if name == "agg" else _make_count()
  return _AGG_CACHE[name]


def _mm(a, w):
  return lax.dot_general(a, w, (((1,), (1,)), ((), ())),
                         preferred_element_type=jnp.float32,
                         precision=lax.Precision.DEFAULT)


def _lrelu(v):
  return jnp.where(v > 0, v, 0.01 * v)


_BLKC = 2048
_NBC = _NP // _BLKC


def _wspec(shape=(_D, _D)):
  return pl.BlockSpec(shape, lambda i: tuple(0 for _ in shape))


def _combine_mean(a0, a1, c0, c1):
  cnt = jnp.maximum(c0[...][:, :1] + c1[...][:, :1], 1.0)
  return (a0[...] + a1[...]) / cnt


def _pre_body(x_ref, w_ref, b_ref, wrn_ref, oh_ref, orn_ref):
  h = _mm(x_ref[...], w_ref[...]) + b_ref[...]
  oh_ref[...] = h
  orn_ref[...] = _mm(h, wrn_ref[...])


_pre = pl.pallas_call(
    _pre_body,
    grid=(_NBC,),
    in_specs=[pl.BlockSpec((_BLKC, _D), lambda i: (i, 0)),
              _wspec(), _wspec((1, _D)), _wspec()],
    out_specs=[pl.BlockSpec((_BLKC, _D), lambda i: (i, 0)),
               pl.BlockSpec((_BLKC, _D), lambda i: (i, 0))],
    out_shape=[jax.ShapeDtypeStruct((_NP, _D), jnp.float32),
               jax.ShapeDtypeStruct((_NP, _D), jnp.float32)],
)


def _comb_body(a0, a1, c0, c1, r_ref, wl_ref, bl_ref, wh_ref, bh_ref,
               wrn_ref, oh_ref, orn_ref):
  mean = _combine_mean(a0, a1, c0, c1)
  s = jnp.maximum(_mm(mean, wl_ref[...]) + bl_ref[...] + r_ref[...], 0.0)
  h = _lrelu(_mm(s, wh_ref[...]) + bh_ref[...])
  oh_ref[...] = h
  orn_ref[...] = _mm(h, wrn_ref[...])


def _make_comb():
  return pl.pallas_call(
      _comb_body,
      grid=(_NBC,),
      in_specs=[pl.BlockSpec((_BLKC, _D), lambda i: (i, 0)),
                pl.BlockSpec((_BLKC, _D), lambda i: (_NBC + i, 0)),
                pl.BlockSpec((_BLKC, 16), lambda i: (i, 0)),
                pl.BlockSpec((_BLKC, 16), lambda i: (_NBC + i, 0)),
                pl.BlockSpec((_BLKC, _D), lambda i: (i, 0)),
                _wspec(), _wspec((1, _D)), _wspec(), _wspec((1, _D)),
                _wspec()],
      out_specs=[pl.BlockSpec((_BLKC, _D), lambda i: (i, 0)),
                 pl.BlockSpec((_BLKC, _D), lambda i: (i, 0))],
      out_shape=[jax.ShapeDtypeStruct((_NP, _D), jnp.float32),
                 jax.ShapeDtypeStruct((_NP, _D), jnp.float32)],
  )


_comb1 = _make_comb()
_comb2 = _make_comb()


def _l3_body(a0, a1, c0, c1, r_ref, b_ref, wl_ref, bl_ref, wo_ref, bo_ref,
             woh_ref, boh_ref, gam_ref, bet_ref, rm_ref, rv_ref, wh1_ref,
             bh1_ref, o_ref, ps_ref, gc_ref):
  @pl.when(pl.program_id(0) == 0)
  def _():
    ps_ref[...] = jnp.zeros_like(ps_ref)
    gc_ref[...] = jnp.zeros_like(gc_ref)

  mean = _combine_mean(a0, a1, c0, c1)
  s = jnp.maximum(_mm(mean, wl_ref[...]) + bl_ref[...] + r_ref[...], 0.0)
  t = _lrelu(_mm(s, wo_ref[...]) + bo_ref[...])
  oh = (b_ref[0] == lax.broadcasted_iota(jnp.int32, (_G, 1), 0)
        ).astype(jnp.float32)
  ps_ref[...] += lax.dot_general(oh, t, (((1,), (0,)), ((), ())),
                                 preferred_element_type=jnp.float32,
                                 precision=lax.Precision.HIGHEST)
  gc_ref[...] += jnp.sum(oh, axis=1, keepdims=True)

  @pl.when(pl.program_id(0) == _NBC - 1)
  def _():
    pooled = ps_ref[...] / jnp.maximum(gc_ref[...][:, :1], 1.0)
    hh = _mm(pooled, woh_ref[...]) + boh_ref[...]
    hh = (hh - rm_ref[...]) / jnp.sqrt(rv_ref[...] + 1e-5) * gam_ref[...] \
        + bet_ref[...]
    hh = _lrelu(hh)
    hw = jnp.sum(hh * wh1_ref[...], axis=1, keepdims=True)
    o_ref[...] = jnp.maximum(hw + bh1_ref[0, 0], 0.0)


_layer3_pool_head = pl.pallas_call(
    _l3_body,
    grid=(_NBC,),
    in_specs=[pl.BlockSpec((_BLKC, _D), lambda i: (i, 0)),
              pl.BlockSpec((_BLKC, _D), lambda i: (_NBC + i, 0)),
              pl.BlockSpec((_BLKC, 16), lambda i: (i, 0)),
              pl.BlockSpec((_BLKC, 16), lambda i: (_NBC + i, 0)),
              pl.BlockSpec((_BLKC, _D), lambda i: (i, 0)),
              pl.BlockSpec((1, 1, _BLKC), lambda i: (i, 0, 0)),
              _wspec(), _wspec((1, _D)), _wspec(), _wspec((1, _D)),
              _wspec(), _wspec((1, _D)), _wspec((1, _D)), _wspec((1, _D)),
              _wspec((1, _D)), _wspec((1, _D)), _wspec((1, _D)),
              _wspec((1, 1))],
    out_specs=pl.BlockSpec((_G, 1), lambda i: (0, 0)),
    out_shape=jax.ShapeDtypeStruct((_G, 1), jnp.float32),
    scratch_shapes=[pltpu.VMEM((_G, _D), jnp.float32),
                    pltpu.VMEM((_G, 1), jnp.float32)],
)


def kernel(x, edge_index, batch, W_pre, b_pre, Wl1, bl1, Wr1, Wl2, bl2, Wr2,
           Wl3, bl3, Wr3, W_hh1, b_hh1, W_hh2, b_hh2, W_oo, b_oo,
           W_oh, b_oh, gamma_h, beta_h, rm_h, rv_h, W_h1, b_h1):
  npad = (_NCHP - _NCH) * _K
  sink = _SINK + (jnp.arange(npad, dtype=jnp.int32) % (_NP - _N))
  src = jnp.concatenate([edge_index[0], sink]).reshape(_NCHP, _K)
  dst = jnp.concatenate([edge_index[1], sink]).reshape(_NCHP, _K)
  xp = jnp.pad(x, ((0, _NP - _N), (0, 0)))
  batch_p = jnp.pad(batch, (0, _NP - _N),
                    constant_values=_G).reshape(_NBC, 1, _BLKC)
  zrow = jnp.zeros((_RPT, _D), jnp.float32)
  ones = jnp.ones((_K, _D), jnp.float32)

  r1 = lambda v: v.reshape(1, -1)

  cnt = _get_agg("count")(dst, ones, zrow)[:, :16]
  h0, rn1 = _pre(xp, W_pre, r1(b_pre), Wr1)
  acc1 = _get_agg("agg")(h0, src, dst, zrow, cnt)
  h1, rn2 = _comb1(acc1, acc1, cnt, cnt, rn1, Wl1, r1(bl1),
                   W_hh1, r1(b_hh1), Wr2)
  acc2 = _get_agg("agg")(h1, src, dst, zrow, cnt)
  h2, rn3 = _comb2(acc2, acc2, cnt, cnt, rn2, Wl2, r1(bl2),
                   W_hh2, r1(b_hh2), Wr3)
  acc3 = _get_agg("agg")(h2, src, dst, zrow, cnt)
  return _layer3_pool_head(acc3, acc3, cnt, cnt, rn3, batch_p,
                           Wl3, r1(bl3), W_oo, r1(b_oo),
                           W_oh, r1(b_oh), r1(gamma_h), r1(beta_h),
                           r1(rm_h), r1(rv_h), W_h1, b_h1.reshape(1, 1))
